# Initial kernel scaffold; baseline (speedup 1.0000x reference)
import jax
import jax.numpy as jnp
from jax.experimental import pallas as pl


def kernel(x, batch, W1, b1, W2, b2):
    raise NotImplementedError("write your pallas kernel here")


# speedup vs baseline: 6.8858x; 6.8858x over previous
import jax
import jax.numpy as jnp
from jax.experimental import pallas as pl
from jax.experimental.pallas import tpu as pltpu

_ROWS = 2000


def _fused_kernel(x_ref, seg_ref, w1_ref, b1_ref, w2_ref, b2_ref,
                  out_ref, m_ref, s_ref):
    i = pl.program_id(0)
    nb = pl.num_programs(0)
    nseg = out_ref.shape[1]

    @pl.when(i == 0)
    def _init():
        m_ref[...] = jnp.full(m_ref.shape, -1e30, jnp.float32)
        s_ref[...] = jnp.zeros(s_ref.shape, jnp.float32)
        out_ref[...] = jnp.zeros(out_ref.shape, jnp.float32)

    x = x_ref[...]
    h = jnp.tanh(jnp.dot(x, w1_ref[...],
                         preferred_element_type=jnp.float32) + b1_ref[...])
    logits = jnp.dot(h, w2_ref[...],
                     preferred_element_type=jnp.float32) + b2_ref[...]

    seg = seg_ref[...]
    lane = jax.lax.broadcasted_iota(jnp.int32, (seg.shape[0], nseg), 1)
    onehot = (seg == lane).astype(jnp.float32)

    masked = jnp.where(onehot > 0.0, logits, jnp.float32(-1e30))
    bmax = jnp.max(masked, axis=0, keepdims=True)
    m_old = m_ref[...]
    m_new = jnp.maximum(m_old, bmax)
    rescale = jnp.exp(m_old - m_new)

    rowmax = jnp.sum(onehot * m_new, axis=1, keepdims=True)
    e = jnp.exp(logits - rowmax)
    p = onehot * e

    m_ref[...] = m_new
    s_ref[...] = s_ref[...] * rescale + jnp.sum(p, axis=0, keepdims=True)
    contrib = jax.lax.dot_general(
        x, p, dimension_numbers=(((0,), (0,)), ((), ())),
        preferred_element_type=jnp.float32)
    out_ref[...] = out_ref[...] * rescale + contrib

    @pl.when(i == nb - 1)
    def _final():
        out_ref[...] = out_ref[...] / (s_ref[...] + 1e-8)


def kernel(x, batch, W1, b1, W2, b2):
    n, d = x.shape
    hidden = W1.shape[1]
    nseg = 64
    rows = _ROWS
    assert n % rows == 0
    grid = n // rows

    out_t = pl.pallas_call(
        _fused_kernel,
        grid=(grid,),
        in_specs=[
            pl.BlockSpec((rows, d), lambda i: (i, 0)),
            pl.BlockSpec((rows, 1), lambda i: (i, 0)),
            pl.BlockSpec((d, hidden), lambda i: (0, 0)),
            pl.BlockSpec((1, hidden), lambda i: (0, 0)),
            pl.BlockSpec((hidden, 1), lambda i: (0, 0)),
            pl.BlockSpec((1, 1), lambda i: (0, 0)),
        ],
        out_specs=pl.BlockSpec((d, nseg), lambda i: (0, 0)),
        out_shape=jax.ShapeDtypeStruct((d, nseg), jnp.float32),
        scratch_shapes=[
            pltpu.VMEM((1, nseg), jnp.float32),
            pltpu.VMEM((1, nseg), jnp.float32),
        ],
    )(x, batch.reshape(n, 1), W1, b1.reshape(1, hidden), W2,
      b2.reshape(1, 1))
    return out_t.T
